# baseline (speedup 1.0000x reference)
import functools

import jax
import jax.numpy as jnp
from jax import lax
from jax.experimental import pallas as pl
from jax.experimental.pallas import tpu as pltpu
from jax.experimental.pallas import tpu_sc as plsc

D = 64
B_TOTAL = 4096 * 200
CHUNK = 128
NUM_CHUNKS = B_TOTAL // CHUNK
SCALE = float(D) ** -0.5
NBUF = 4


def _make_sc_kernel(chunk_base0, num_chunks_half):
    info = plsc.get_sparse_core_info()
    ns = info.num_subcores
    cpw = num_chunks_half // ns
    nsteps = cpw // NBUF

    mesh = plsc.VectorSubcoreMesh(
        core_axis_name="c", subcore_axis_name="s", num_cores=1)

    scratch = [
        pltpu.VMEM((cpw, CHUNK), jnp.int32),
        pltpu.VMEM((NBUF, CHUNK, D), jnp.float32),
        pltpu.VMEM((NBUF, CHUNK, D), jnp.float32),
    ] + [pltpu.SemaphoreType.DMA] * (2 * NBUF)

    @functools.partial(
        pl.kernel,
        out_type=jax.ShapeDtypeStruct((num_chunks_half * CHUNK, D),
                                      jnp.float32),
        mesh=mesh,
        scratch_types=scratch,
        compiler_params=pltpu.CompilerParams(use_tc_tiling_on_sc=False),
    )
    def emb_kernel(idx_hbm, table_hbm, out_hbm, idx_v, in_v, sc_v, *sems):
        gsem = sems[:NBUF]
        ssem = sems[NBUF:]
        wid = lax.axis_index("s")
        chunk_base = chunk_base0 + wid * cpw
        out_base = wid * cpw
        pltpu.sync_copy(idx_hbm.at[pl.ds(chunk_base, cpw)], idx_v)

        def start_gather(b, c):
            pltpu.async_copy(table_hbm.at[idx_v.at[c]], in_v.at[b], gsem[b])

        def start_store(b, c):
            out_start = (out_base + c) * CHUNK
            pltpu.async_copy(sc_v.at[b], out_hbm.at[pl.ds(out_start, CHUNK)],
                             ssem[b])

        def wait_gather(b):
            pltpu.make_async_copy(table_hbm.at[idx_v.at[0]], in_v.at[b],
                                  gsem[b]).wait()

        def wait_store(b):
            pltpu.make_async_copy(sc_v.at[b], out_hbm.at[pl.ds(0, CHUNK)],
                                  ssem[b]).wait()

        def scale(b):
            def row(i, carry):
                for j in range(D // 16):
                    s = pl.ds(j * 16, 16)
                    sc_v[b, i, s] = in_v[b, i, s] * SCALE
                return carry

            lax.fori_loop(0, CHUNK, row, 0, unroll=4)

        for b in range(NBUF):
            start_gather(b, b)
        for b in range(NBUF):
            wait_gather(b)
            scale(b)
            start_store(b, b)
            start_gather(b, NBUF + b)

        def step(g0, carry):
            for b in range(NBUF):
                c = g0 * NBUF + b
                wait_gather(b)
                wait_store(b)
                scale(b)
                start_store(b, c)
                start_gather(b, c + NBUF)
            return carry

        lax.fori_loop(1, nsteps - 1, step, 0)

        for b in range(NBUF):
            c = (nsteps - 1) * NBUF + b
            wait_gather(b)
            wait_store(b)
            scale(b)
            start_store(b, c)
        for b in range(NBUF):
            wait_store(b)

    return emb_kernel


_HALF = NUM_CHUNKS // 2
_emb_lo = _make_sc_kernel(0, _HALF)
_emb_hi = _make_sc_kernel(_HALF, _HALF)


@jax.jit
def kernel(x, table):
    idx = x.reshape(NUM_CHUNKS, CHUNK).astype(jnp.int32)
    lo = _emb_lo(idx, table)
    hi = _emb_hi(idx, table)
    out = jnp.concatenate([lo, hi], axis=0)
    return out.reshape(x.shape + (D,))
